# baseline (speedup 1.0000x reference)
import functools

import jax
import jax.numpy as jnp
from jax import lax
from jax.experimental import pallas as pl
from jax.experimental.pallas import tpu as pltpu
from jax.experimental.pallas import tpu_sc as plsc

_BM = 10000


def _mm_relu(x, w):
    n, d = x.shape

    def body(x_ref, w_ref, o_ref):
        o_ref[...] = jnp.maximum(
            jnp.dot(x_ref[...], w_ref[...], preferred_element_type=jnp.float32), 0.0
        )

    return pl.pallas_call(
        body,
        grid=(n // _BM,),
        in_specs=[
            pl.BlockSpec((_BM, d), lambda i: (i, 0)),
            pl.BlockSpec((d, d), lambda i: (0, 0)),
        ],
        out_specs=pl.BlockSpec((_BM, d), lambda i: (i, 0)),
        out_shape=jax.ShapeDtypeStruct((n, d), jnp.float32),
    )(x, w)


def _sc_segment_sum(h, edge_flat):
    n, d = h.shape
    e = edge_flat.shape[0] // 2
    info = plsc.get_sparse_core_info()
    nc, ns = info.num_cores, info.num_subcores
    nw = nc * ns
    assert e % nw == 0
    epw = e // nw
    k = 80
    assert epw % k == 0
    ch = epw // k
    rpt = -(-n // (ns * k)) * k
    npad = ns * rpt

    mesh = plsc.VectorSubcoreMesh(core_axis_name="c", subcore_axis_name="s")
    bin_ = 25
    assert ch % bin_ == 0
    nblk = ch // bin_

    @functools.partial(
        pl.kernel,
        out_type=jax.ShapeDtypeStruct((nc * npad, d), jnp.float32),
        mesh=mesh,
    scratch_types=[
            pltpu.VMEM((epw,), jnp.int32),
            pltpu.VMEM((4, k), jnp.int32),
            pltpu.VMEM((3, k, d), jnp.float32),
            pltpu.VMEM_SHARED((npad, d), jnp.float32),
            pltpu.SemaphoreType.DMA,
            pltpu.SemaphoreType.DMA,
            pltpu.SemaphoreType.DMA,
            pltpu.SemaphoreType.DMA,
            pltpu.SemaphoreType.DMA,
            pltpu.SemaphoreType.DMA,
            pltpu.SemaphoreType.DMA,
            pltpu.SemaphoreType.DMA,
            pltpu.SemaphoreType.DMA,
        ],
    )
    def seg(ei_hbm, h_hbm, out_hbm, src_v, dbuf, rows, acc,
            isem0, isem1, isem2, isem3, gsem0, gsem1, gsem2, ssem0, ssem1):
        cid = lax.axis_index("c")
        sid = lax.axis_index("s")
        wid = sid * nc + cid
        isems = (isem0, isem1, isem2, isem3)
        gsems = (gsem0, gsem1, gsem2)
        ssems = (ssem0, ssem1)
        base = wid * epw

        ip0 = pltpu.async_copy(ei_hbm.at[pl.ds(base, epw)], src_v, gsem0)

        zv = jnp.zeros((16,), jnp.float32)

        def zrow(r, carry):
            for c in range(d // 16):
                rows[0, r, pl.ds(c * 16, 16)] = zv
            return carry

        lax.fori_loop(0, k, zrow, 0)
        zcopies = [
            pltpu.async_copy(rows.at[0], acc.at[pl.ds(sid * rpt + j * k, k)],
                             isem0)
            for j in range(rpt // k)
        ]
        ip0.wait()
        for c in zcopies:
            c.wait()
        plsc.subcore_barrier()

        def issue_dst(cj, s):
            return pltpu.async_copy(
                ei_hbm.at[pl.ds(e + base + cj * k, k)], dbuf.at[s], isems[s])

        def issue_gather(cj, b):
            return pltpu.async_copy(
                h_hbm.at[src_v.at[pl.ds(cj * k, k)]], rows.at[b], gsems[b])

        def block(blk, carry):
            cbase = blk * bin_
            di = [issue_dst(cbase + 0, 0), issue_dst(cbase + 1, 1),
                  issue_dst(cbase + 2, 2), None]
            dg = [issue_gather(cbase + 0, 0), issue_gather(cbase + 1, 1),
                  None]
            ds = [None, None]
            for j in range(bin_):
                rb = j % 3
                qb = j % 4
                sb = j % 2
                if j + 2 < bin_:
                    if ds[1 - sb] is not None:
                        ds[1 - sb].wait()
                        ds[1 - sb] = None
                    dg[(j + 2) % 3] = issue_gather(cbase + j + 2, (j + 2) % 3)
                if j + 3 < bin_:
                    di[(j + 3) % 4] = issue_dst(cbase + j + 3, (j + 3) % 4)
                dg[rb].wait()
                di[qb].wait()
                if ds[sb] is not None:
                    ds[sb].wait()
                ds[sb] = pltpu.async_copy(rows.at[rb], acc.at[dbuf.at[qb]],
                                          ssems[sb], add=True)
            for q in ds:
                if q is not None:
                    q.wait()
            return carry

        lax.fori_loop(0, nblk, block, 0)
        plsc.subcore_barrier()

        pltpu.sync_copy(
            acc.at[pl.ds(sid * rpt, rpt)],
            out_hbm.at[pl.ds(cid * npad + sid * rpt, rpt)],
        )

    return seg(edge_flat, h), npad


def _tail(x, parts, w_o1, b_o1, w_g1, b_g1, w_g2, b_g2, w_g3, b_g3):
    n, d = x.shape

    def body(x_ref, p_ref, wo_ref, wg1_ref, wg2_ref, wg3_ref,
             bo_ref, bg1_ref, bg2_ref, bg3_ref, o_ref):
        xb = x_ref[...]
        agg = p_ref[0] + p_ref[1]
        ret = jnp.dot(xb, wo_ref[...], preferred_element_type=jnp.float32) \
            + bo_ref[...] + agg
        t = jnp.maximum(
            jnp.dot(xb, wg1_ref[...], preferred_element_type=jnp.float32)
            + bg1_ref[...]
            + jnp.dot(agg, wg2_ref[...], preferred_element_type=jnp.float32)
            + bg2_ref[...],
            0.0,
        )
        g = jnp.dot(t, wg3_ref[...], preferred_element_type=jnp.float32) + bg3_ref[...]
        gate = 1.0 / (1.0 + jnp.exp(-g))
        o_ref[...] = ret * gate + xb * (1.0 - gate)

    wspec = pl.BlockSpec((d, d), lambda i: (0, 0))
    bspec = pl.BlockSpec((1, d), lambda i: (0, 0))
    return pl.pallas_call(
        body,
        grid=(n // _BM,),
        in_specs=[
            pl.BlockSpec((_BM, d), lambda i: (i, 0)),
            pl.BlockSpec((2, _BM, d), lambda i: (0, i, 0)),
            wspec, wspec, wspec, wspec,
            bspec, bspec, bspec, bspec,
        ],
        out_specs=pl.BlockSpec((_BM, d), lambda i: (i, 0)),
        out_shape=jax.ShapeDtypeStruct((n, d), jnp.float32),
    )(x, parts, w_o1, w_g1, w_g2, w_g3,
      b_o1.reshape(1, d), b_g1.reshape(1, d), b_g2.reshape(1, d), b_g3.reshape(1, d))


def kernel(x, edge_index, Wm, W_o1, b_o1, W_g1, b_g1, W_g2, b_g2, W_g3, b_g3):
    n, d = x.shape
    h = _mm_relu(x, Wm)
    parts, npad = _sc_segment_sum(h, edge_index.reshape(-1))
    return _tail(x, parts.reshape(2, npad, d), W_o1, b_o1, W_g1, b_g1,
                 W_g2, b_g2, W_g3, b_g3)


# speedup vs baseline: 1.0117x; 1.0117x over previous
import functools

import jax
import jax.numpy as jnp
from jax import lax
from jax.experimental import pallas as pl
from jax.experimental.pallas import tpu as pltpu
from jax.experimental.pallas import tpu_sc as plsc

_BM = 5000


def _mm_relu(x, w):
    n, d = x.shape

    def body(x_ref, w_ref, o_ref):
        o_ref[...] = jnp.maximum(
            jnp.dot(x_ref[...], w_ref[...], preferred_element_type=jnp.float32), 0.0
        )

    return pl.pallas_call(
        body,
        grid=(n // _BM,),
        in_specs=[
            pl.BlockSpec((_BM, d), lambda i: (i, 0)),
            pl.BlockSpec((d, d), lambda i: (0, 0)),
        ],
        out_specs=pl.BlockSpec((_BM, d), lambda i: (i, 0)),
        out_shape=jax.ShapeDtypeStruct((n, d), jnp.float32),
    )(x, w)


def _sc_segment_sum(h, edge_flat):
    n, d = h.shape
    e = edge_flat.shape[0] // 2
    info = plsc.get_sparse_core_info()
    nc, ns = info.num_cores, info.num_subcores
    nw = nc * ns
    assert e % nw == 0
    epw = e // nw
    k = 80
    assert epw % k == 0
    ch = epw // k
    rpt = -(-n // (ns * k)) * k
    npad = ns * rpt

    mesh = plsc.VectorSubcoreMesh(core_axis_name="c", subcore_axis_name="s")
    bin_ = 25
    assert ch % bin_ == 0
    nblk = ch // bin_

    @functools.partial(
        pl.kernel,
        out_type=jax.ShapeDtypeStruct((nc * npad, d), jnp.float32),
        mesh=mesh,
    scratch_types=[
            pltpu.VMEM((epw,), jnp.int32),
            pltpu.VMEM((4, k), jnp.int32),
            pltpu.VMEM((3, k, d), jnp.float32),
            pltpu.VMEM_SHARED((npad, d), jnp.float32),
            pltpu.SemaphoreType.DMA,
            pltpu.SemaphoreType.DMA,
            pltpu.SemaphoreType.DMA,
            pltpu.SemaphoreType.DMA,
            pltpu.SemaphoreType.DMA,
            pltpu.SemaphoreType.DMA,
            pltpu.SemaphoreType.DMA,
            pltpu.SemaphoreType.DMA,
            pltpu.SemaphoreType.DMA,
        ],
    )
    def seg(ei_hbm, h_hbm, out_hbm, src_v, dbuf, rows, acc,
            isem0, isem1, isem2, isem3, gsem0, gsem1, gsem2, ssem0, ssem1):
        cid = lax.axis_index("c")
        sid = lax.axis_index("s")
        wid = sid * nc + cid
        isems = (isem0, isem1, isem2, isem3)
        gsems = (gsem0, gsem1, gsem2)
        ssems = (ssem0, ssem1)
        base = wid * epw

        ip0 = pltpu.async_copy(ei_hbm.at[pl.ds(base, epw)], src_v, gsem0)

        zv = jnp.zeros((16,), jnp.float32)

        def zrow(r, carry):
            for c in range(d // 16):
                rows[0, r, pl.ds(c * 16, 16)] = zv
            return carry

        lax.fori_loop(0, k, zrow, 0)
        zcopies = [
            pltpu.async_copy(rows.at[0], acc.at[pl.ds(sid * rpt + j * k, k)],
                             isem0)
            for j in range(rpt // k)
        ]
        ip0.wait()
        for c in zcopies:
            c.wait()
        plsc.subcore_barrier()

        def issue_dst(cj, s):
            return pltpu.async_copy(
                ei_hbm.at[pl.ds(e + base + cj * k, k)], dbuf.at[s], isems[s])

        def issue_gather(cj, b):
            return pltpu.async_copy(
                h_hbm.at[src_v.at[pl.ds(cj * k, k)]], rows.at[b], gsems[b])

        def block(blk, carry):
            cbase = blk * bin_
            di = [issue_dst(cbase + 0, 0), issue_dst(cbase + 1, 1),
                  issue_dst(cbase + 2, 2), None]
            dg = [issue_gather(cbase + 0, 0), issue_gather(cbase + 1, 1),
                  None]
            ds = [None, None]
            for j in range(bin_):
                rb = j % 3
                qb = j % 4
                sb = j % 2
                if j + 2 < bin_:
                    if ds[1 - sb] is not None:
                        ds[1 - sb].wait()
                        ds[1 - sb] = None
                    dg[(j + 2) % 3] = issue_gather(cbase + j + 2, (j + 2) % 3)
                if j + 3 < bin_:
                    di[(j + 3) % 4] = issue_dst(cbase + j + 3, (j + 3) % 4)
                dg[rb].wait()
                di[qb].wait()
                if ds[sb] is not None:
                    ds[sb].wait()
                ds[sb] = pltpu.async_copy(rows.at[rb], acc.at[dbuf.at[qb]],
                                          ssems[sb], add=True)
            for q in ds:
                if q is not None:
                    q.wait()
            return carry

        lax.fori_loop(0, nblk, block, 0)
        plsc.subcore_barrier()

        pltpu.sync_copy(
            acc.at[pl.ds(sid * rpt, rpt)],
            out_hbm.at[pl.ds(cid * npad + sid * rpt, rpt)],
        )

    return seg(edge_flat, h), npad


def _tail(x, parts, w_o1, b_o1, w_g1, b_g1, w_g2, b_g2, w_g3, b_g3):
    n, d = x.shape

    def body(x_ref, p_ref, wo_ref, wg1_ref, wg2_ref, wg3_ref,
             bo_ref, bg1_ref, bg2_ref, bg3_ref, o_ref):
        xb = x_ref[...]
        agg = p_ref[0] + p_ref[1]
        ret = jnp.dot(xb, wo_ref[...], preferred_element_type=jnp.float32) \
            + bo_ref[...] + agg
        t = jnp.maximum(
            jnp.dot(xb, wg1_ref[...], preferred_element_type=jnp.float32)
            + bg1_ref[...]
            + jnp.dot(agg, wg2_ref[...], preferred_element_type=jnp.float32)
            + bg2_ref[...],
            0.0,
        )
        g = jnp.dot(t, wg3_ref[...], preferred_element_type=jnp.float32) + bg3_ref[...]
        gate = 1.0 / (1.0 + jnp.exp(-g))
        o_ref[...] = ret * gate + xb * (1.0 - gate)

    wspec = pl.BlockSpec((d, d), lambda i: (0, 0))
    bspec = pl.BlockSpec((1, d), lambda i: (0, 0))
    return pl.pallas_call(
        body,
        grid=(n // _BM,),
        in_specs=[
            pl.BlockSpec((_BM, d), lambda i: (i, 0)),
            pl.BlockSpec((2, _BM, d), lambda i: (0, i, 0)),
            wspec, wspec, wspec, wspec,
            bspec, bspec, bspec, bspec,
        ],
        out_specs=pl.BlockSpec((_BM, d), lambda i: (i, 0)),
        out_shape=jax.ShapeDtypeStruct((n, d), jnp.float32),
    )(x, parts, w_o1, w_g1, w_g2, w_g3,
      b_o1.reshape(1, d), b_g1.reshape(1, d), b_g2.reshape(1, d), b_g3.reshape(1, d))


def kernel(x, edge_index, Wm, W_o1, b_o1, W_g1, b_g1, W_g2, b_g2, W_g3, b_g3):
    n, d = x.shape
    h = _mm_relu(x, Wm)
    parts, npad = _sc_segment_sum(h, edge_index.reshape(-1))
    return _tail(x, parts.reshape(2, npad, d), W_o1, b_o1, W_g1, b_g1,
                 W_g2, b_g2, W_g3, b_g3)
